# Initial kernel scaffold; baseline (speedup 1.0000x reference)
import jax
import jax.numpy as jnp
from jax.experimental import pallas as pl


def kernel(x, weights):
    raise NotImplementedError("write your pallas kernel here")


# speedup vs baseline: 6.5093x; 6.5093x over previous
import functools

import jax
import jax.numpy as jnp
from jax import lax
from jax.experimental import pallas as pl
from jax.experimental.pallas import tpu as pltpu
from jax.experimental.pallas import tpu_sc as plsc

B = 1024
F = 1024
OUT = 256
R = 16
T = 8
L = 16
NTILES = 32
WPT = B // NTILES // 32
ROWS = B // NTILES


def _sc_body(x_hbm, w_hbm, out_hbm, xbuf, wbuf, col, obuf):
    wid = lax.axis_index("s") * 2 + lax.axis_index("c")
    base_row = wid * ROWS

    pltpu.sync_copy(x_hbm.at[pl.ds(base_row, ROWS)], xbuf)
    pltpu.sync_copy(w_hbm, wbuf)

    col[pl.ds(0, L)] = jnp.full((L,), -1, jnp.int32)

    def pack_body(fg, carry):
        acc = jnp.zeros((L,), jnp.int32)
        for b in range(ROWS):
            acc = acc | (xbuf[b, pl.ds(fg * L, L)] << b)
        col[pl.ds(1 + fg * L, L)] = acc
        col[pl.ds(1 + F + fg * L, L)] = ~acc
        return carry

    lax.fori_loop(0, F // L, pack_body, 0)

    def og_body(og, carry):
        obase = og * L
        or_acc = jnp.zeros((L,), jnp.int32)
        for r in range(R):
            acc = jnp.full((L,), -1, jnp.int32)
            nz = jnp.zeros((L,), jnp.int32)
            for t in range(T):
                tidx = wbuf[r, t, pl.ds(obase, L)]
                g = plsc.load_gather(col, [tidx])
                acc = acc & g
                nz = nz | tidx
            or_acc = or_acc | jnp.where(nz != 0, acc, 0)
        for b in range(ROWS):
            obuf[b, pl.ds(obase, L)] = (or_acc >> b) & 1
        return carry

    lax.fori_loop(0, OUT // L, og_body, 0)

    pltpu.sync_copy(obuf, out_hbm.at[pl.ds(base_row, ROWS)])


def kernel(x, weights):
    w_t = jnp.transpose(weights, (1, 2, 0))
    mesh = plsc.VectorSubcoreMesh(core_axis_name="c", subcore_axis_name="s")
    f = functools.partial(
        pl.kernel,
        out_type=jax.ShapeDtypeStruct((B, OUT), jnp.int32),
        mesh=mesh,
        compiler_params=pltpu.CompilerParams(needs_layout_passes=False),
        scratch_types=[
            pltpu.VMEM((ROWS, F), jnp.int32),
            pltpu.VMEM((R, T, OUT), jnp.int32),
            pltpu.VMEM((1 + 2 * F + 15, ), jnp.int32),
            pltpu.VMEM((ROWS, OUT), jnp.int32),
        ],
    )(_sc_body)
    return f(x, w_t).astype(bool)


# speedup vs baseline: 7.3933x; 1.1358x over previous
import functools

import jax
import jax.numpy as jnp
from jax import lax
from jax.experimental import pallas as pl
from jax.experimental.pallas import tpu as pltpu
from jax.experimental.pallas import tpu_sc as plsc

B = 1024
F = 1024
OUT = 256
R = 16
T = 8
L = 16
NTILES = 32
ROWS = B // NTILES


def _sc_body(x_hbm, w_hbm, out_hbm, xbuf, wbuf, col, obuf, sem_x, sem_w):
    wid = lax.axis_index("s") * 2 + lax.axis_index("c")
    base_row = wid * ROWS

    cp_w = pltpu.async_copy(w_hbm, wbuf, sem_w)
    cp_x = pltpu.async_copy(x_hbm.at[pl.ds(base_row, ROWS)], xbuf, sem_x)

    col[pl.ds(0, L)] = jnp.full((L,), -1, jnp.int32)

    cp_x.wait()

    def pack_body(fg, carry):
        acc = jnp.zeros((L,), jnp.int32)
        for b in range(ROWS):
            acc = acc | (xbuf[b, pl.ds(fg * L, L)] << b)
        col[pl.ds(1 + fg * L, L)] = acc
        col[pl.ds(1 + F + fg * L, L)] = ~acc
        return carry

    lax.fori_loop(0, F // L, pack_body, 0)

    cp_w.wait()

    def og_body(og, carry):
        obase = og * L

        def r_body(r, or_acc):
            acc = jnp.full((L,), -1, jnp.int32)
            nz = jnp.zeros((L,), jnp.int32)
            for tp in range(T // 2):
                ab = plsc.bitcast(wbuf[r, tp, pl.ds(obase, L)], jnp.int16)
                ia, ib = plsc.unpack(ab, format=plsc.PackFormat.INTERLEAVED)
                acc = acc & plsc.load_gather(col, [ia])
                acc = acc & plsc.load_gather(col, [ib])
                nz = nz | ia | ib
            return or_acc | jnp.where(nz != 0, acc, 0)

        or_acc = lax.fori_loop(0, R, r_body, jnp.zeros((L,), jnp.int32),
                               unroll=2)
        for b in range(ROWS):
            obuf[b, pl.ds(obase, L)] = (or_acc >> b) & 1
        return carry

    lax.fori_loop(0, OUT // L, og_body, 0)

    pltpu.sync_copy(obuf, out_hbm.at[pl.ds(base_row, ROWS)])


def kernel(x, weights):
    w_t = jnp.transpose(weights, (1, 2, 0))
    w_t = w_t.reshape(R, T // 2, 2, OUT)
    w16 = w_t[:, :, 0, :] | (w_t[:, :, 1, :] << 16)
    mesh = plsc.VectorSubcoreMesh(core_axis_name="c", subcore_axis_name="s")
    f = functools.partial(
        pl.kernel,
        out_type=jax.ShapeDtypeStruct((B, OUT), jnp.int32),
        mesh=mesh,
        compiler_params=pltpu.CompilerParams(needs_layout_passes=False),
        scratch_types=[
            pltpu.VMEM((ROWS, F), jnp.int32),
            pltpu.VMEM((R, T // 2, OUT), jnp.int32),
            pltpu.VMEM((1 + 2 * F + 15, ), jnp.int32),
            pltpu.VMEM((ROWS, OUT), jnp.int32),
            pltpu.SemaphoreType.DMA,
            pltpu.SemaphoreType.DMA,
        ],
    )(_sc_body)
    return f(x, w16).astype(bool)
